# baseline (speedup 1.0000x reference)
import functools

import jax
import jax.numpy as jnp
from jax import lax
from jax.experimental import pallas as pl
from jax.experimental.pallas import tpu as pltpu
from jax.experimental.pallas import tpu_sc as plsc

_LANES = 16


@functools.partial(jax.jit, static_argnums=(3, 4, 5, 6))
def _sc_lookup(etab, mtab, igbp, B, D, Dp, L):
    info = plsc.get_sparse_core_info()
    NC, NS = info.num_cores, info.num_subcores
    NW = NC * NS
    bw = B // NW
    n_idx_rows = bw // 128

    mesh = plsc.VectorSubcoreMesh(core_axis_name="c", subcore_axis_name="s")

    @functools.partial(
        pl.kernel,
        mesh=mesh,
        compiler_params=pltpu.CompilerParams(use_tc_tiling_on_sc=False),
        out_type=[
            jax.ShapeDtypeStruct((B, Dp), jnp.float32),
            jax.ShapeDtypeStruct((B, L), jnp.uint8),
        ],
        scratch_types=[
            pltpu.VMEM((bw,), jnp.int32),
            pltpu.VMEM((n_idx_rows, 128), jnp.int32),
            pltpu.VMEM((bw, Dp), jnp.float32),
            pltpu.VMEM((bw, L), jnp.uint8),
            pltpu.SemaphoreType.DMA,
            pltpu.SemaphoreType.DMA,
        ],
    )
    def body(etab_h, mtab_h, igbp_h, emb_out, mask_out,
             ig_v, idx_v, erows, mrows, sem_e, sem_m):
        num_codes = etab_h.shape[0] - 1
        wid = lax.axis_index("s") * NC + lax.axis_index("c")
        base = wid * bw

        pltpu.sync_copy(igbp_h.at[pl.ds(base, bw)], ig_v)

        for j in range(n_idx_rows):
            for t in range(128 // _LANES):
                x = ig_v[pl.ds((j * 128 + t * _LANES), _LANES)]
                valid = (x >= 0) & (x < num_codes)
                idx_v[j, pl.ds(t * _LANES, _LANES)] = jnp.where(
                    valid, x, num_codes)

        copies = []
        for j in range(n_idx_rows):
            copies.append(pltpu.async_copy(
                etab_h.at[idx_v.at[j]], erows.at[pl.ds(j * 128, 128)], sem_e))
            copies.append(pltpu.async_copy(
                mtab_h.at[idx_v.at[j]], mrows.at[pl.ds(j * 128, 128)], sem_m))
        for c in copies:
            c.wait()

        pltpu.sync_copy(erows, emb_out.at[pl.ds(base, bw)])
        pltpu.sync_copy(mrows, mask_out.at[pl.ds(base, bw)])

    return body(etab, mtab, igbp)


def kernel(igbp, predictor_values, emb_table):
    B = igbp.shape[0]
    L = predictor_values.shape[1]
    num_codes, D = emb_table.shape
    Dp = (D + _LANES - 1) // _LANES * _LANES
    etab = jnp.zeros((num_codes + 1, Dp), emb_table.dtype)
    etab = etab.at[:num_codes, :D].set(emb_table)
    mtab = jnp.broadcast_to(
        (jnp.arange(num_codes + 1) >= num_codes)[:, None].astype(jnp.uint8),
        (num_codes + 1, L))
    emb_p, mask_u8 = _sc_lookup(etab, mtab, igbp, B, D, Dp, L)
    return emb_p[:, None, :D], mask_u8.astype(jnp.bool_)[:, :, None]


# speedup vs baseline: 1.4822x; 1.4822x over previous
import functools

import jax
import jax.numpy as jnp
from jax import lax
from jax.experimental import pallas as pl
from jax.experimental.pallas import tpu as pltpu
from jax.experimental.pallas import tpu_sc as plsc

_LANES = 16
_MASK_PACK = 4


@functools.partial(jax.jit, static_argnums=(2, 3, 4, 5))
def _sc_lookup(tab_f, igbp, B, NCODES, D, L):
    info = plsc.get_sparse_core_info()
    NC, NS = info.num_cores, info.num_subcores
    NW = NC * NS
    bw = B // NW
    n_chunks = bw // _LANES
    mw = L // _MASK_PACK

    mesh = plsc.VectorSubcoreMesh(core_axis_name="c", subcore_axis_name="s")

    @functools.partial(
        pl.kernel,
        mesh=mesh,
        compiler_params=pltpu.CompilerParams(
            use_tc_tiling_on_sc=False, needs_layout_passes=False),
        out_type=[
            jax.ShapeDtypeStruct((B * D,), jnp.float32),
            jax.ShapeDtypeStruct((B * mw,), jnp.int32),
        ],
        scratch_types=[
            pltpu.VMEM((NCODES * D,), jnp.float32),
            pltpu.VMEM((bw,), jnp.int32),
            pltpu.VMEM((bw * D,), jnp.float32),
            pltpu.VMEM((bw * mw,), jnp.int32),
        ],
    )
    def body(tab_h, igbp_h, emb_out, mask_out, tab_v, ig_v, erows, mrows):
        wid = lax.axis_index("s") * NC + lax.axis_index("c")
        base = wid * bw

        pltpu.sync_copy(tab_h, tab_v)
        pltpu.sync_copy(igbp_h.at[pl.ds(base, bw)], ig_v)

        lanes = lax.iota(jnp.int32, _LANES)

        def chunk(i, carry):
            ig = ig_v[pl.ds(i * _LANES, _LANES)]
            valid = (ig >= 0) & (ig < NCODES)
            valid_f = valid.astype(jnp.float32)
            sidx = jnp.where(valid, ig, 0)
            addr = sidx * D
            out0 = (i * _LANES + lanes) * D
            for c in range(D):
                col = plsc.load_gather(tab_v, [addr + c]) * valid_f
                plsc.store_scatter(erows, [out0 + c], col)
            nv_words = jnp.where(valid, 0, 0x01010101)
            m0 = (i * _LANES + lanes) * mw
            for w in range(mw):
                plsc.store_scatter(mrows, [m0 + w], nv_words)
            return carry

        lax.fori_loop(0, n_chunks, chunk, 0)

        pltpu.sync_copy(erows, emb_out.at[pl.ds(base * D, bw * D)])
        pltpu.sync_copy(mrows, mask_out.at[pl.ds(base * mw, bw * mw)])

    return body(tab_f, igbp)


def kernel(igbp, predictor_values, emb_table):
    B = igbp.shape[0]
    L = predictor_values.shape[1]
    num_codes, D = emb_table.shape
    emb_f, mask_w = _sc_lookup(
        emb_table.reshape(-1), igbp, B, num_codes, D, L)
    emb = emb_f.reshape(B, 1, D)
    mask = (
        lax.bitcast_convert_type(mask_w, jnp.uint8)
        .reshape(B, L, 1)
        .astype(jnp.bool_)
    )
    return emb, mask


# speedup vs baseline: 2.1174x; 1.4285x over previous
import functools

import jax
import jax.numpy as jnp
from jax import lax
from jax.experimental import pallas as pl
from jax.experimental.pallas import tpu as pltpu
from jax.experimental.pallas import tpu_sc as plsc

_LANES = 16
_MASK_PACK = 4


@functools.partial(jax.jit, static_argnums=(2, 3, 4, 5))
def _sc_lookup(tab_f, igbp, B, NCODES, D, L):
    info = plsc.get_sparse_core_info()
    NC, NS = info.num_cores, info.num_subcores
    NW = NC * NS
    bw = B // NW
    n_chunks = bw // _LANES
    mwt = bw // _MASK_PACK

    mesh = plsc.VectorSubcoreMesh(core_axis_name="c", subcore_axis_name="s")

    @functools.partial(
        pl.kernel,
        mesh=mesh,
        compiler_params=pltpu.CompilerParams(
            use_tc_tiling_on_sc=False, needs_layout_passes=False),
        out_type=[
            jax.ShapeDtypeStruct((D, B), jnp.float32),
            jax.ShapeDtypeStruct((L, B // _MASK_PACK), jnp.int32),
        ],
        scratch_types=[
            pltpu.VMEM((NCODES * D,), jnp.float32),
            pltpu.VMEM((bw,), jnp.int32),
            pltpu.VMEM((D, bw), jnp.float32),
            pltpu.VMEM((L, mwt), jnp.int32),
        ],
    )
    def body(tab_h, igbp_h, emb_out, mask_out, tab_v, ig_v, ecols, mrep):
        wid = lax.axis_index("s") * NC + lax.axis_index("c")
        base = wid * bw

        pltpu.sync_copy(tab_h, tab_v)
        pltpu.sync_copy(igbp_h.at[pl.ds(base, bw)], ig_v)

        lanes = lax.iota(jnp.int32, _LANES)

        def chunk(i, carry):
            off = i * _LANES
            ig = ig_v[pl.ds(off, _LANES)]
            valid = (ig >= 0) & (ig < NCODES)
            valid_f = valid.astype(jnp.float32)
            addr = jnp.where(valid, ig, 0) * D
            for d in range(D):
                col = plsc.load_gather(tab_v, [addr + d]) * valid_f
                ecols[d, pl.ds(off, _LANES)] = col
            return carry

        lax.fori_loop(0, n_chunks, chunk, 0)

        def mchunk(m, carry):
            w = jnp.zeros((_LANES,), jnp.int32)
            for j in range(_MASK_PACK):
                ig = plsc.load_gather(
                    ig_v, [m * (_LANES * _MASK_PACK) + _MASK_PACK * lanes + j])
                nv = ((ig < 0) | (ig >= NCODES)).astype(jnp.int32)
                w = w | (nv << (8 * j))
            for r in range(L):
                mrep[r, pl.ds(m * _LANES, _LANES)] = w
            return carry

        lax.fori_loop(0, mwt // _LANES, mchunk, 0)

        pltpu.sync_copy(ecols, emb_out.at[:, pl.ds(base, bw)])
        pltpu.sync_copy(mrep, mask_out.at[:, pl.ds(wid * mwt, mwt)])

    return body(tab_f, igbp)


def kernel(igbp, predictor_values, emb_table):
    B = igbp.shape[0]
    L = predictor_values.shape[1]
    num_codes, D = emb_table.shape
    emb_t, mask_w = _sc_lookup(
        emb_table.reshape(-1), igbp, B, num_codes, D, L)
    emb = emb_t.T[:, None, :]
    mask = (
        lax.bitcast_convert_type(mask_w, jnp.uint8)
        .reshape(L, B)
        .T[:, :, None]
        .astype(jnp.bool_)
    )
    return emb, mask
